# baseline (speedup 1.0000x reference)
import functools

import jax
import jax.numpy as jnp
from jax import lax
from jax.experimental import pallas as pl
from jax.experimental.pallas import tpu as pltpu
from jax.experimental.pallas import tpu_sc as plsc

_K = 24
_KPAD = 32
_R = 128
_T = 2048
_DPAD = 128


def _sc_gather_rows(table_pad, idx_grouped, n_rows_out):
    info = plsc.get_sparse_core_info()
    nc, ns = info.num_cores, info.num_subcores
    nw = nc * ns
    b_per_w = n_rows_out // nw
    n_chunks = b_per_w // 128
    mesh = plsc.VectorSubcoreMesh(core_axis_name="c", subcore_axis_name="s")

    @functools.partial(
        pl.kernel,
        mesh=mesh,
        out_type=jax.ShapeDtypeStruct((n_rows_out, _DPAD), jnp.float32),
        scratch_types=[
            pltpu.VMEM((n_chunks, 128), jnp.int32),
            pltpu.VMEM((b_per_w, _DPAD), jnp.float32),
            pltpu.SemaphoreType.DMA,
        ],
    )
    def k(table_hbm, idx_hbm, out_hbm, idx_v, rows_v, sem):
        wid = lax.axis_index("s") * nc + lax.axis_index("c")
        pltpu.sync_copy(idx_hbm.at[wid], idx_v)
        for ci in range(n_chunks):
            pltpu.async_copy(
                table_hbm.at[idx_v.at[ci]],
                rows_v.at[pl.ds(ci * 128, 128)],
                sem,
            ).wait()
        pltpu.sync_copy(rows_v, out_hbm.at[pl.ds(wid * b_per_w, b_per_w)])

    return k(table_pad, idx_grouped)


def _sc_edge_d2(xt, yt, zt, src, dst):
    info = plsc.get_sparse_core_info()
    nc, ns = info.num_cores, info.num_subcores
    nw = nc * ns
    n_edges = src.shape[0]
    n_nodes = xt.shape[0]
    e_per_w = n_edges // nw
    n_vec = e_per_w // 16
    mesh = plsc.VectorSubcoreMesh(core_axis_name="c", subcore_axis_name="s")

    @functools.partial(
        pl.kernel,
        mesh=mesh,
        out_type=jax.ShapeDtypeStruct((n_edges,), jnp.float32),
        compiler_params=pltpu.CompilerParams(needs_layout_passes=False),
        scratch_types=[
            pltpu.VMEM((n_nodes,), jnp.float32),
            pltpu.VMEM((n_nodes,), jnp.float32),
            pltpu.VMEM((n_nodes,), jnp.float32),
            pltpu.VMEM((e_per_w,), jnp.int32),
            pltpu.VMEM((e_per_w,), jnp.int32),
            pltpu.VMEM((e_per_w,), jnp.float32),
        ],
    )
    def k(xt_hbm, yt_hbm, zt_hbm, src_hbm, dst_hbm, out_hbm,
          x_v, y_v, z_v, src_v, dst_v, d2_v):
        wid = lax.axis_index("s") * nc + lax.axis_index("c")
        base = wid * e_per_w
        pltpu.sync_copy(xt_hbm, x_v)
        pltpu.sync_copy(yt_hbm, y_v)
        pltpu.sync_copy(zt_hbm, z_v)
        pltpu.sync_copy(src_hbm.at[pl.ds(base, e_per_w)], src_v)
        pltpu.sync_copy(dst_hbm.at[pl.ds(base, e_per_w)], dst_v)

        @pl.loop(0, n_vec)
        def body(j):
            o = j * 16
            isrc = src_v[pl.ds(o, 16)]
            idst = dst_v[pl.ds(o, 16)]
            dx = plsc.load_gather(x_v, [isrc]) - plsc.load_gather(x_v, [idst])
            dy = plsc.load_gather(y_v, [isrc]) - plsc.load_gather(y_v, [idst])
            dz = plsc.load_gather(z_v, [isrc]) - plsc.load_gather(z_v, [idst])
            d2_v[pl.ds(o, 16)] = (dx * dx + dy * dy) + dz * dz

        pltpu.sync_copy(d2_v, out_hbm.at[pl.ds(base, e_per_w)])

    return k(xt, yt, zt, src, dst)


def _tc_body(meta_ref, rows_ref, colst_ref, idx_ref, sel, cid):
    b = pl.program_id(0)
    lo = meta_ref[b, 0]
    nt = meta_ref[b, 1]

    rows = rows_ref[...]
    rx = rows[:, 0:1]
    ry = rows[:, 1:2]
    rz = rows[:, 2:3]
    rb = rows[:, 3:4]
    rsq = rows[:, 4:5]
    lhs_z = jnp.where(
        lax.broadcasted_iota(jnp.int32, (_R, 8), 1) < 3, rows,
        jnp.float32(0.0))
    row_ids = b * _R + lax.broadcasted_iota(jnp.int32, (_R, 1), 0)

    inf = jnp.float32(3.0e38)
    bigi = jnp.int32(2**30)

    def build(t, _):
        w = pl.multiple_of(t * _T, _T)
        c = pl.multiple_of(lo + w, 128)
        colt = colst_ref[:, pl.ds(c, _T)]
        rhs_z = jnp.where(
            lax.broadcasted_iota(jnp.int32, (8, _T), 0) < 3, colt,
            jnp.float32(0.0))
        gram = lax.dot_general(
            lhs_z, rhs_z, (((1,), (0,)), ((), ())),
            preferred_element_type=jnp.float32,
            precision=lax.Precision.DEFAULT)
        d2 = (rsq + colt[4:5, :]) - jnp.float32(2.0) * gram
        d2 = jnp.maximum(d2, jnp.float32(0.0))
        colid = c + lax.broadcasted_iota(jnp.int32, (_R, _T), 1)
        d2 = jnp.where(rb == colt[3:4, :], d2, jnp.float32(1e30))
        d2 = jnp.where(colid == row_ids, jnp.float32(-1.0), d2)
        sel[:, pl.ds(w, _T)] = d2
        cid[:, pl.ds(w, _T)] = colid.astype(jnp.float32)
        return 0

    lax.fori_loop(0, nt, build, 0)

    idxs = [row_ids.astype(jnp.float32)]
    for kk in range(1, _K):
        prev = idxs[-1]

        def step(t, carry, prev=prev):
            bv, bi = carry
            w = pl.multiple_of(t * _T, _T)
            colid = cid[:, pl.ds(w, _T)]
            m = jnp.where(colid == prev, inf, sel[:, pl.ds(w, _T)])
            sel[:, pl.ds(w, _T)] = m
            tv = jnp.min(m, axis=1, keepdims=True)
            ti = jnp.min(jnp.where(m == tv, colid, inf), axis=1,
                         keepdims=True)
            upd = tv < bv
            return jnp.where(upd, tv, bv), jnp.where(upd, ti, bi)

        _, bi = lax.fori_loop(
            0, nt, step,
            (jnp.full((_R, 1), inf, jnp.float32),
             jnp.full((_R, 1), inf, jnp.float32)),
        )
        idxs.append(bi)

    pad = _KPAD - _K
    allif = jnp.concatenate(idxs + [jnp.zeros((_R, pad), jnp.float32)], axis=1)
    idx_ref[...] = allif.astype(jnp.int32)


def _tc_knn_topk(meta, rows_arr, colst, m, interpret=False):
    nb = m // _R
    return pl.pallas_call(
        _tc_body,
        grid_spec=pltpu.PrefetchScalarGridSpec(
            num_scalar_prefetch=1,
            grid=(nb,),
            in_specs=[
                pl.BlockSpec((_R, 8), lambda b, meta: (b, 0)),
                pl.BlockSpec((8, m), lambda b, meta: (0, 0)),
            ],
            out_specs=[
                pl.BlockSpec((_R, _KPAD), lambda b, meta: (b, 0)),
            ],
            scratch_shapes=[pltpu.VMEM((_R, m), jnp.float32),
                            pltpu.VMEM((_R, m), jnp.float32)],
        ),
        out_shape=[
            jax.ShapeDtypeStruct((m, _KPAD), jnp.int32),
        ],
        compiler_params=pltpu.CompilerParams(
            dimension_semantics=("parallel",)),
        interpret=interpret,
    )(meta, rows_arr, colst)


def _window_meta(batch, m):
    bfirst = batch[:: _R]
    blast = batch[_R - 1 :: _R]
    lo = jnp.searchsorted(batch, bfirst, side="left").astype(jnp.int32)
    hi = jnp.searchsorted(batch, blast, side="right").astype(jnp.int32)
    lo_a = (lo // 128) * 128
    nt = (hi - lo_a + _T - 1) // _T
    lo_a = jnp.minimum(lo_a, m - nt * _T)
    return jnp.stack([lo_a, nt], axis=1).astype(jnp.int32)


def kernel(x, pos, edge_index, edge_weight, batch, perm, score, i):
    m = perm.shape[0]

    table_pad = jnp.pad(pos, ((0, 0), (0, _DPAD - pos.shape[1])))
    idx_grouped = perm.reshape(32, -1, 128)
    pos_p16 = _sc_gather_rows(table_pad, idx_grouped, m)
    pos_p = pos_p16[:, :3]

    batch_f = batch.astype(jnp.float32)
    sq = jnp.sum(pos_p * pos_p, axis=1)
    rows_arr = jnp.concatenate(
        [pos_p, batch_f[:, None], sq[:, None], jnp.zeros((m, 3), jnp.float32)],
        axis=1,
    )
    colst = rows_arr.T
    meta = _window_meta(batch, m)

    (idx_out,) = _tc_knn_topk(meta, rows_arr, colst, m)
    nbr = idx_out[:, :_K]

    src = nbr.reshape(-1)
    dst = jnp.repeat(jnp.arange(m, dtype=jnp.int32), _K)
    ei_new = jnp.stack([src, dst], axis=0)
    d2e = _sc_edge_d2(pos_p[:, 0], pos_p[:, 1], pos_p[:, 2], src, dst)
    dist = jnp.sqrt(jnp.maximum(d2e, jnp.float32(1e-12)))
    ew_new = dist / jnp.maximum(jnp.max(dist), jnp.float32(1e-12))
    return (x, pos_p, ei_new, ew_new, batch, perm, score)


# speedup vs baseline: 1.0350x; 1.0350x over previous
import functools

import jax
import jax.numpy as jnp
from jax import lax
from jax.experimental import pallas as pl
from jax.experimental.pallas import tpu as pltpu
from jax.experimental.pallas import tpu_sc as plsc

_K = 24
_KPAD = 32
_R = 256
_T = 2048
_DPAD = 128


def _sc_gather_rows(table_pad, idx_grouped, n_rows_out):
    info = plsc.get_sparse_core_info()
    nc, ns = info.num_cores, info.num_subcores
    nw = nc * ns
    b_per_w = n_rows_out // nw
    n_chunks = b_per_w // 128
    mesh = plsc.VectorSubcoreMesh(core_axis_name="c", subcore_axis_name="s")

    @functools.partial(
        pl.kernel,
        mesh=mesh,
        out_type=jax.ShapeDtypeStruct((n_rows_out, _DPAD), jnp.float32),
        scratch_types=[
            pltpu.VMEM((n_chunks, 128), jnp.int32),
            pltpu.VMEM((b_per_w, _DPAD), jnp.float32),
            pltpu.SemaphoreType.DMA,
        ],
    )
    def k(table_hbm, idx_hbm, out_hbm, idx_v, rows_v, sem):
        wid = lax.axis_index("s") * nc + lax.axis_index("c")
        pltpu.sync_copy(idx_hbm.at[wid], idx_v)
        for ci in range(n_chunks):
            pltpu.async_copy(
                table_hbm.at[idx_v.at[ci]],
                rows_v.at[pl.ds(ci * 128, 128)],
                sem,
            ).wait()
        pltpu.sync_copy(rows_v, out_hbm.at[pl.ds(wid * b_per_w, b_per_w)])

    return k(table_pad, idx_grouped)


def _sc_edge_d2(xt, yt, zt, src, dst):
    info = plsc.get_sparse_core_info()
    nc, ns = info.num_cores, info.num_subcores
    nw = nc * ns
    n_edges = src.shape[0]
    n_nodes = xt.shape[0]
    e_per_w = n_edges // nw
    n_vec = e_per_w // 16
    mesh = plsc.VectorSubcoreMesh(core_axis_name="c", subcore_axis_name="s")

    @functools.partial(
        pl.kernel,
        mesh=mesh,
        out_type=jax.ShapeDtypeStruct((n_edges,), jnp.float32),
        compiler_params=pltpu.CompilerParams(needs_layout_passes=False),
        scratch_types=[
            pltpu.VMEM((n_nodes,), jnp.float32),
            pltpu.VMEM((n_nodes,), jnp.float32),
            pltpu.VMEM((n_nodes,), jnp.float32),
            pltpu.VMEM((e_per_w,), jnp.int32),
            pltpu.VMEM((e_per_w,), jnp.int32),
            pltpu.VMEM((e_per_w,), jnp.float32),
        ],
    )
    def k(xt_hbm, yt_hbm, zt_hbm, src_hbm, dst_hbm, out_hbm,
          x_v, y_v, z_v, src_v, dst_v, d2_v):
        wid = lax.axis_index("s") * nc + lax.axis_index("c")
        base = wid * e_per_w
        pltpu.sync_copy(xt_hbm, x_v)
        pltpu.sync_copy(yt_hbm, y_v)
        pltpu.sync_copy(zt_hbm, z_v)
        pltpu.sync_copy(src_hbm.at[pl.ds(base, e_per_w)], src_v)
        pltpu.sync_copy(dst_hbm.at[pl.ds(base, e_per_w)], dst_v)

        @pl.loop(0, n_vec)
        def body(j):
            o = j * 16
            isrc = src_v[pl.ds(o, 16)]
            idst = dst_v[pl.ds(o, 16)]
            dx = plsc.load_gather(x_v, [isrc]) - plsc.load_gather(x_v, [idst])
            dy = plsc.load_gather(y_v, [isrc]) - plsc.load_gather(y_v, [idst])
            dz = plsc.load_gather(z_v, [isrc]) - plsc.load_gather(z_v, [idst])
            d2_v[pl.ds(o, 16)] = (dx * dx + dy * dy) + dz * dz

        pltpu.sync_copy(d2_v, out_hbm.at[pl.ds(base, e_per_w)])

    return k(xt, yt, zt, src, dst)


def _tc_body(meta_ref, rows_ref, colst_ref, idx_ref, sel, cid):
    b = pl.program_id(0)
    lo = meta_ref[b, 0]
    nt = meta_ref[b, 1]

    rows = rows_ref[...]
    rx = rows[:, 0:1]
    ry = rows[:, 1:2]
    rz = rows[:, 2:3]
    rb = rows[:, 3:4]
    rsq = rows[:, 4:5]
    lhs_z = jnp.where(
        lax.broadcasted_iota(jnp.int32, (_R, 8), 1) < 3, rows,
        jnp.float32(0.0))
    row_ids = b * _R + lax.broadcasted_iota(jnp.int32, (_R, 1), 0)

    inf = jnp.float32(3.0e38)
    bigi = jnp.int32(2**30)

    def build(t, carry):
        bv, bi = carry
        w = pl.multiple_of(t * _T, _T)
        c = pl.multiple_of(lo + w, 128)
        colt = colst_ref[:, pl.ds(c, _T)]
        rhs_z = jnp.where(
            lax.broadcasted_iota(jnp.int32, (8, _T), 0) < 3, colt,
            jnp.float32(0.0))
        gram = lax.dot_general(
            lhs_z, rhs_z, (((1,), (0,)), ((), ())),
            preferred_element_type=jnp.float32,
            precision=lax.Precision.DEFAULT)
        d2 = (rsq + colt[4:5, :]) - jnp.float32(2.0) * gram
        d2 = jnp.maximum(d2, jnp.float32(0.0))
        colid = c + lax.broadcasted_iota(jnp.int32, (_R, _T), 1)
        d2 = jnp.where(rb == colt[3:4, :], d2, jnp.float32(1e30))
        cidf = colid.astype(jnp.float32)
        d2 = jnp.where(colid == row_ids, inf, d2)
        sel[:, pl.ds(w, _T)] = d2
        cid[:, pl.ds(w, _T)] = cidf
        tv = jnp.min(d2, axis=1, keepdims=True)
        ti = jnp.min(jnp.where(d2 == tv, cidf, inf), axis=1, keepdims=True)
        upd = tv < bv
        return jnp.where(upd, tv, bv), jnp.where(upd, ti, bi)

    _, bi1 = lax.fori_loop(
        0, nt, build,
        (jnp.full((_R, 1), inf, jnp.float32),
         jnp.full((_R, 1), inf, jnp.float32)),
    )

    idxs = [row_ids.astype(jnp.float32), bi1]
    for kk in range(2, _K):
        prev = idxs[-1]

        def step(t, carry, prev=prev):
            bv, bi = carry
            w = pl.multiple_of(t * _T, _T)
            colid = cid[:, pl.ds(w, _T)]
            m = jnp.where(colid == prev, inf, sel[:, pl.ds(w, _T)])
            sel[:, pl.ds(w, _T)] = m
            tv = jnp.min(m, axis=1, keepdims=True)
            ti = jnp.min(jnp.where(m == tv, colid, inf), axis=1,
                         keepdims=True)
            upd = tv < bv
            return jnp.where(upd, tv, bv), jnp.where(upd, ti, bi)

        _, bi = lax.fori_loop(
            0, nt, step,
            (jnp.full((_R, 1), inf, jnp.float32),
             jnp.full((_R, 1), inf, jnp.float32)),
        )
        idxs.append(bi)

    pad = _KPAD - _K
    allif = jnp.concatenate(idxs + [jnp.zeros((_R, pad), jnp.float32)], axis=1)
    idx_ref[...] = allif.astype(jnp.int32)


def _tc_knn_topk(meta, rows_arr, colst, m, interpret=False):
    nb = m // _R
    return pl.pallas_call(
        _tc_body,
        grid_spec=pltpu.PrefetchScalarGridSpec(
            num_scalar_prefetch=1,
            grid=(nb,),
            in_specs=[
                pl.BlockSpec((_R, 8), lambda b, meta: (b, 0)),
                pl.BlockSpec((8, m), lambda b, meta: (0, 0)),
            ],
            out_specs=[
                pl.BlockSpec((_R, _KPAD), lambda b, meta: (b, 0)),
            ],
            scratch_shapes=[pltpu.VMEM((_R, m), jnp.float32),
                            pltpu.VMEM((_R, m), jnp.float32)],
        ),
        out_shape=[
            jax.ShapeDtypeStruct((m, _KPAD), jnp.int32),
        ],
        compiler_params=pltpu.CompilerParams(
            dimension_semantics=("parallel",)),
        interpret=interpret,
    )(meta, rows_arr, colst)


def _window_meta(batch, m):
    bfirst = batch[:: _R]
    blast = batch[_R - 1 :: _R]
    lo = jnp.searchsorted(batch, bfirst, side="left").astype(jnp.int32)
    hi = jnp.searchsorted(batch, blast, side="right").astype(jnp.int32)
    lo_a = (lo // 128) * 128
    nt = (hi - lo_a + _T - 1) // _T
    lo_a = jnp.minimum(lo_a, m - nt * _T)
    return jnp.stack([lo_a, nt], axis=1).astype(jnp.int32)


def kernel(x, pos, edge_index, edge_weight, batch, perm, score, i):
    m = perm.shape[0]

    table_pad = jnp.pad(pos, ((0, 0), (0, _DPAD - pos.shape[1])))
    idx_grouped = perm.reshape(32, -1, 128)
    pos_p16 = _sc_gather_rows(table_pad, idx_grouped, m)
    pos_p = pos_p16[:, :3]

    batch_f = batch.astype(jnp.float32)
    sq = jnp.sum(pos_p * pos_p, axis=1)
    rows_arr = jnp.concatenate(
        [pos_p, batch_f[:, None], sq[:, None], jnp.zeros((m, 3), jnp.float32)],
        axis=1,
    )
    colst = rows_arr.T
    meta = _window_meta(batch, m)

    (idx_out,) = _tc_knn_topk(meta, rows_arr, colst, m)
    nbr = idx_out[:, :_K]

    src = nbr.reshape(-1)
    dst = jnp.repeat(jnp.arange(m, dtype=jnp.int32), _K)
    ei_new = jnp.stack([src, dst], axis=0)
    d2e = _sc_edge_d2(pos_p[:, 0], pos_p[:, 1], pos_p[:, 2], src, dst)
    dist = jnp.sqrt(jnp.maximum(d2e, jnp.float32(1e-12)))
    ew_new = dist / jnp.maximum(jnp.max(dist), jnp.float32(1e-12))
    return (x, pos_p, ei_new, ew_new, batch, perm, score)


# speedup vs baseline: 1.3749x; 1.3284x over previous
import functools

import jax
import jax.numpy as jnp
from jax import lax
from jax.experimental import pallas as pl
from jax.experimental.pallas import tpu as pltpu
from jax.experimental.pallas import tpu_sc as plsc

_K = 24
_KPAD = 32
_R = 256
_T = 1280
_DPAD = 128


def _sc_gather_rows(table_pad, idx_grouped, n_rows_out):
    info = plsc.get_sparse_core_info()
    nc, ns = info.num_cores, info.num_subcores
    nw = nc * ns
    b_per_w = n_rows_out // nw
    n_chunks = b_per_w // 128
    mesh = plsc.VectorSubcoreMesh(core_axis_name="c", subcore_axis_name="s")

    @functools.partial(
        pl.kernel,
        mesh=mesh,
        out_type=jax.ShapeDtypeStruct((n_rows_out, _DPAD), jnp.float32),
        scratch_types=[
            pltpu.VMEM((n_chunks, 128), jnp.int32),
            pltpu.VMEM((b_per_w, _DPAD), jnp.float32),
            pltpu.SemaphoreType.DMA,
        ],
    )
    def k(table_hbm, idx_hbm, out_hbm, idx_v, rows_v, sem):
        wid = lax.axis_index("s") * nc + lax.axis_index("c")
        pltpu.sync_copy(idx_hbm.at[wid], idx_v)
        for ci in range(n_chunks):
            pltpu.async_copy(
                table_hbm.at[idx_v.at[ci]],
                rows_v.at[pl.ds(ci * 128, 128)],
                sem,
            ).wait()
        pltpu.sync_copy(rows_v, out_hbm.at[pl.ds(wid * b_per_w, b_per_w)])

    return k(table_pad, idx_grouped)


def _sc_edge_d2(xt, yt, zt, src, dst):
    info = plsc.get_sparse_core_info()
    nc, ns = info.num_cores, info.num_subcores
    nw = nc * ns
    n_edges = src.shape[0]
    n_nodes = xt.shape[0]
    e_per_w = n_edges // nw
    n_vec = e_per_w // 16
    mesh = plsc.VectorSubcoreMesh(core_axis_name="c", subcore_axis_name="s")

    @functools.partial(
        pl.kernel,
        mesh=mesh,
        out_type=jax.ShapeDtypeStruct((n_edges,), jnp.float32),
        compiler_params=pltpu.CompilerParams(needs_layout_passes=False),
        scratch_types=[
            pltpu.VMEM((n_nodes,), jnp.float32),
            pltpu.VMEM((n_nodes,), jnp.float32),
            pltpu.VMEM((n_nodes,), jnp.float32),
            pltpu.VMEM((e_per_w,), jnp.int32),
            pltpu.VMEM((e_per_w,), jnp.int32),
            pltpu.VMEM((e_per_w,), jnp.float32),
        ],
    )
    def k(xt_hbm, yt_hbm, zt_hbm, src_hbm, dst_hbm, out_hbm,
          x_v, y_v, z_v, src_v, dst_v, d2_v):
        wid = lax.axis_index("s") * nc + lax.axis_index("c")
        base = wid * e_per_w
        pltpu.sync_copy(xt_hbm, x_v)
        pltpu.sync_copy(yt_hbm, y_v)
        pltpu.sync_copy(zt_hbm, z_v)
        pltpu.sync_copy(src_hbm.at[pl.ds(base, e_per_w)], src_v)
        pltpu.sync_copy(dst_hbm.at[pl.ds(base, e_per_w)], dst_v)

        @pl.loop(0, n_vec)
        def body(j):
            o = j * 16
            isrc = src_v[pl.ds(o, 16)]
            idst = dst_v[pl.ds(o, 16)]
            dx = plsc.load_gather(x_v, [isrc]) - plsc.load_gather(x_v, [idst])
            dy = plsc.load_gather(y_v, [isrc]) - plsc.load_gather(y_v, [idst])
            dz = plsc.load_gather(z_v, [isrc]) - plsc.load_gather(z_v, [idst])
            d2_v[pl.ds(o, 16)] = (dx * dx + dy * dy) + dz * dz

        pltpu.sync_copy(d2_v, out_hbm.at[pl.ds(base, e_per_w)])

    return k(xt, yt, zt, src, dst)


def _tc_body(meta_ref, rows_ref, colst_ref, idx_ref, sel, cid):
    b = pl.program_id(0)
    lo = meta_ref[b, 0]
    nt = meta_ref[b, 1]

    rows = rows_ref[...]
    rx = rows[:, 0:1]
    ry = rows[:, 1:2]
    rz = rows[:, 2:3]
    rb = rows[:, 3:4]
    rsq = rows[:, 4:5]
    lhs_z = jnp.where(
        lax.broadcasted_iota(jnp.int32, (_R, 8), 1) < 3, rows,
        jnp.float32(0.0))
    row_ids = b * _R + lax.broadcasted_iota(jnp.int32, (_R, 1), 0)

    inf = jnp.float32(3.0e38)
    bigi = jnp.int32(2**30)

    def build(t, carry):
        bv, bi = carry
        mcols = colst_ref.shape[1]
        w = pl.multiple_of(t * _T, _T)
        c = pl.multiple_of(jnp.minimum(lo + w, mcols - _T), 128)
        colt = colst_ref[:, pl.ds(c, _T)]
        rhs_z = jnp.where(
            lax.broadcasted_iota(jnp.int32, (8, _T), 0) < 3, colt,
            jnp.float32(0.0))
        gram = lax.dot_general(
            lhs_z, rhs_z, (((1,), (0,)), ((), ())),
            preferred_element_type=jnp.float32,
            precision=lax.Precision.DEFAULT)
        d2 = (rsq + colt[4:5, :]) - jnp.float32(2.0) * gram
        d2 = jnp.maximum(d2, jnp.float32(0.0))
        colid = c + lax.broadcasted_iota(jnp.int32, (_R, _T), 1)
        d2 = jnp.where(rb == colt[3:4, :], d2, jnp.float32(1e30))
        cidf = colid.astype(jnp.float32)
        d2 = jnp.where(colid == row_ids, inf, d2)
        sel[:, pl.ds(w, _T)] = d2
        cid[:, pl.ds(w, _T)] = cidf
        tv = jnp.min(d2, axis=1, keepdims=True)
        ti = jnp.min(jnp.where(d2 == tv, cidf, inf), axis=1, keepdims=True)
        upd = tv < bv
        return jnp.where(upd, tv, bv), jnp.where(upd, ti, bi)

    _, bi1 = lax.fori_loop(
        0, nt, build,
        (jnp.full((_R, 1), inf, jnp.float32),
         jnp.full((_R, 1), inf, jnp.float32)),
    )

    idxs = [row_ids.astype(jnp.float32), bi1]
    for kk in range(2, _K):
        prev = idxs[-1]

        def step(t, carry, prev=prev):
            bv, bi = carry
            w = pl.multiple_of(t * _T, _T)
            colid = cid[:, pl.ds(w, _T)]
            m = jnp.where(colid == prev, inf, sel[:, pl.ds(w, _T)])
            sel[:, pl.ds(w, _T)] = m
            tv = jnp.min(m, axis=1, keepdims=True)
            ti = jnp.min(jnp.where(m == tv, colid, inf), axis=1,
                         keepdims=True)
            upd = tv < bv
            return jnp.where(upd, tv, bv), jnp.where(upd, ti, bi)

        _, bi = lax.fori_loop(
            0, nt, step,
            (jnp.full((_R, 1), inf, jnp.float32),
             jnp.full((_R, 1), inf, jnp.float32)),
        )
        idxs.append(bi)

    pad = _KPAD - _K
    allif = jnp.concatenate(idxs + [jnp.zeros((_R, pad), jnp.float32)], axis=1)
    idx_ref[...] = allif.astype(jnp.int32)


def _tc_knn_topk(meta, rows_arr, colst, m, interpret=False):
    nb = m // _R
    return pl.pallas_call(
        _tc_body,
        grid_spec=pltpu.PrefetchScalarGridSpec(
            num_scalar_prefetch=1,
            grid=(nb,),
            in_specs=[
                pl.BlockSpec((_R, 8), lambda b, meta: (b, 0)),
                pl.BlockSpec((8, m), lambda b, meta: (0, 0)),
            ],
            out_specs=[
                pl.BlockSpec((_R, _KPAD), lambda b, meta: (b, 0)),
            ],
            scratch_shapes=[pltpu.VMEM((_R, -(-m // _T) * _T), jnp.float32),
                            pltpu.VMEM((_R, -(-m // _T) * _T), jnp.float32)],
        ),
        out_shape=[
            jax.ShapeDtypeStruct((m, _KPAD), jnp.int32),
        ],
        compiler_params=pltpu.CompilerParams(
            dimension_semantics=("parallel",)),
        interpret=interpret,
    )(meta, rows_arr, colst)


def _window_meta(batch, m):
    bfirst = batch[:: _R]
    blast = batch[_R - 1 :: _R]
    lo = jnp.searchsorted(batch, bfirst, side="left").astype(jnp.int32)
    hi = jnp.searchsorted(batch, blast, side="right").astype(jnp.int32)
    lo_a = (lo // 128) * 128
    nt = (hi - lo_a + _T - 1) // _T
    return jnp.stack([lo_a, nt], axis=1).astype(jnp.int32)


def kernel(x, pos, edge_index, edge_weight, batch, perm, score, i):
    m = perm.shape[0]

    table_pad = jnp.pad(pos, ((0, 0), (0, _DPAD - pos.shape[1])))
    idx_grouped = perm.reshape(32, -1, 128)
    pos_p16 = _sc_gather_rows(table_pad, idx_grouped, m)
    pos_p = pos_p16[:, :3]

    batch_f = batch.astype(jnp.float32)
    sq = jnp.sum(pos_p * pos_p, axis=1)
    rows_arr = jnp.concatenate(
        [pos_p, batch_f[:, None], sq[:, None], jnp.zeros((m, 3), jnp.float32)],
        axis=1,
    )
    colst = rows_arr.T
    meta = _window_meta(batch, m)

    (idx_out,) = _tc_knn_topk(meta, rows_arr, colst, m)
    nbr = idx_out[:, :_K]

    src = nbr.reshape(-1)
    dst = jnp.repeat(jnp.arange(m, dtype=jnp.int32), _K)
    ei_new = jnp.stack([src, dst], axis=0)
    d2e = _sc_edge_d2(pos_p[:, 0], pos_p[:, 1], pos_p[:, 2], src, dst)
    dist = jnp.sqrt(jnp.maximum(d2e, jnp.float32(1e-12)))
    ew_new = dist / jnp.maximum(jnp.max(dist), jnp.float32(1e-12))
    return (x, pos_p, ei_new, ew_new, batch, perm, score)


# speedup vs baseline: 1.4123x; 1.0272x over previous
import functools

import jax
import jax.numpy as jnp
from jax import lax
from jax.experimental import pallas as pl
from jax.experimental.pallas import tpu as pltpu
from jax.experimental.pallas import tpu_sc as plsc

_K = 24
_KPAD = 32
_R = 256
_T = 1280
_DPAD = 128


def _sc_gather_rows(table_pad, idx_grouped, n_rows_out):
    info = plsc.get_sparse_core_info()
    nc, ns = info.num_cores, info.num_subcores
    nw = nc * ns
    b_per_w = n_rows_out // nw
    n_chunks = b_per_w // 128
    mesh = plsc.VectorSubcoreMesh(core_axis_name="c", subcore_axis_name="s")

    @functools.partial(
        pl.kernel,
        mesh=mesh,
        out_type=jax.ShapeDtypeStruct((n_rows_out, _DPAD), jnp.float32),
        scratch_types=[
            pltpu.VMEM((n_chunks, 128), jnp.int32),
            pltpu.VMEM((b_per_w, _DPAD), jnp.float32),
            pltpu.SemaphoreType.DMA,
        ],
    )
    def k(table_hbm, idx_hbm, out_hbm, idx_v, rows_v, sem):
        wid = lax.axis_index("s") * nc + lax.axis_index("c")
        pltpu.sync_copy(idx_hbm.at[wid], idx_v)
        for ci in range(n_chunks):
            pltpu.async_copy(
                table_hbm.at[idx_v.at[ci]],
                rows_v.at[pl.ds(ci * 128, 128)],
                sem,
            ).wait()
        pltpu.sync_copy(rows_v, out_hbm.at[pl.ds(wid * b_per_w, b_per_w)])

    return k(table_pad, idx_grouped)


def _sc_edge_d2(xt, yt, zt, src, dst):
    info = plsc.get_sparse_core_info()
    nc, ns = info.num_cores, info.num_subcores
    nw = nc * ns
    n_edges = src.shape[0]
    n_nodes = xt.shape[0]
    e_per_w = n_edges // nw
    n_vec = e_per_w // 16
    mesh = plsc.VectorSubcoreMesh(core_axis_name="c", subcore_axis_name="s")

    @functools.partial(
        pl.kernel,
        mesh=mesh,
        out_type=jax.ShapeDtypeStruct((n_edges,), jnp.float32),
        compiler_params=pltpu.CompilerParams(needs_layout_passes=False),
        scratch_types=[
            pltpu.VMEM((n_nodes,), jnp.float32),
            pltpu.VMEM((n_nodes,), jnp.float32),
            pltpu.VMEM((n_nodes,), jnp.float32),
            pltpu.VMEM((e_per_w,), jnp.int32),
            pltpu.VMEM((e_per_w,), jnp.int32),
            pltpu.VMEM((e_per_w,), jnp.float32),
        ],
    )
    def k(xt_hbm, yt_hbm, zt_hbm, src_hbm, dst_hbm, out_hbm,
          x_v, y_v, z_v, src_v, dst_v, d2_v):
        wid = lax.axis_index("s") * nc + lax.axis_index("c")
        base = wid * e_per_w
        pltpu.sync_copy(xt_hbm, x_v)
        pltpu.sync_copy(yt_hbm, y_v)
        pltpu.sync_copy(zt_hbm, z_v)
        pltpu.sync_copy(src_hbm.at[pl.ds(base, e_per_w)], src_v)
        pltpu.sync_copy(dst_hbm.at[pl.ds(base, e_per_w)], dst_v)

        @pl.loop(0, n_vec)
        def body(j):
            o = j * 16
            isrc = src_v[pl.ds(o, 16)]
            idst = dst_v[pl.ds(o, 16)]
            dx = plsc.load_gather(x_v, [isrc]) - plsc.load_gather(x_v, [idst])
            dy = plsc.load_gather(y_v, [isrc]) - plsc.load_gather(y_v, [idst])
            dz = plsc.load_gather(z_v, [isrc]) - plsc.load_gather(z_v, [idst])
            d2_v[pl.ds(o, 16)] = (dx * dx + dy * dy) + dz * dz

        pltpu.sync_copy(d2_v, out_hbm.at[pl.ds(base, e_per_w)])

    return k(xt, yt, zt, src, dst)


def _tc_body(meta_ref, rows_ref, colst_ref, idx_ref, sel, cid):
    b = pl.program_id(0)
    lo = meta_ref[b, 0]
    nt = meta_ref[b, 1]

    rows = rows_ref[...]
    rx = rows[:, 0:1]
    ry = rows[:, 1:2]
    rz = rows[:, 2:3]
    rb = rows[:, 3:4]
    rsq = rows[:, 4:5]
    lhs_z = jnp.where(
        lax.broadcasted_iota(jnp.int32, (_R, 8), 1) < 3, rows,
        jnp.float32(0.0))
    row_ids = b * _R + lax.broadcasted_iota(jnp.int32, (_R, 1), 0)

    inf = jnp.float32(3.0e38)
    bigi = jnp.int32(2**30)

    def build(t, carry):
        bv, bi = carry
        mcols = colst_ref.shape[1]
        w = pl.multiple_of(t * _T, _T)
        c = pl.multiple_of(jnp.minimum(lo + w, mcols - _T), 128)
        colt = colst_ref[:, pl.ds(c, _T)]
        rhs_z = jnp.where(
            lax.broadcasted_iota(jnp.int32, (8, _T), 0) < 3, colt,
            jnp.float32(0.0))
        gram = lax.dot_general(
            lhs_z, rhs_z, (((1,), (0,)), ((), ())),
            preferred_element_type=jnp.float32,
            precision=lax.Precision.DEFAULT)
        d2 = (rsq + colt[4:5, :]) - jnp.float32(2.0) * gram
        d2 = jnp.maximum(d2, jnp.float32(0.0))
        colid = c + lax.broadcasted_iota(jnp.int32, (_R, _T), 1)
        d2 = jnp.where(rb == colt[3:4, :], d2, jnp.float32(1e30))
        cidf = colid.astype(jnp.float32)
        d2 = jnp.where(colid == row_ids, inf, d2)
        sel[:, pl.ds(w, _T)] = d2
        cid[:, pl.ds(w, _T)] = cidf
        tv = jnp.min(d2, axis=1, keepdims=True)
        ti = jnp.min(jnp.where(d2 == tv, cidf, inf), axis=1, keepdims=True)
        upd = tv < bv
        return jnp.where(upd, tv, bv), jnp.where(upd, ti, bi)

    carry0 = build(0, (jnp.full((_R, 1), inf, jnp.float32),
                       jnp.full((_R, 1), inf, jnp.float32)))
    _, bi1 = lax.fori_loop(1, nt, build, carry0)

    idxs = [row_ids.astype(jnp.float32), bi1]
    for kk in range(2, _K):
        prev = idxs[-1]

        def step(t, carry, prev=prev):
            bv, bi = carry
            w = pl.multiple_of(t * _T, _T)
            colid = cid[:, pl.ds(w, _T)]
            m = jnp.where(colid == prev, inf, sel[:, pl.ds(w, _T)])
            sel[:, pl.ds(w, _T)] = m
            tv = jnp.min(m, axis=1, keepdims=True)
            ti = jnp.min(jnp.where(m == tv, colid, inf), axis=1,
                         keepdims=True)
            upd = tv < bv
            return jnp.where(upd, tv, bv), jnp.where(upd, ti, bi)

        carry0 = step(0, (jnp.full((_R, 1), inf, jnp.float32),
                          jnp.full((_R, 1), inf, jnp.float32)))
        _, bi = lax.fori_loop(1, nt, step, carry0)
        idxs.append(bi)

    pad = _KPAD - _K
    allif = jnp.concatenate(idxs + [jnp.zeros((_R, pad), jnp.float32)], axis=1)
    idx_ref[...] = allif.astype(jnp.int32)


def _tc_knn_topk(meta, rows_arr, colst, m, interpret=False):
    nb = m // _R
    return pl.pallas_call(
        _tc_body,
        grid_spec=pltpu.PrefetchScalarGridSpec(
            num_scalar_prefetch=1,
            grid=(nb,),
            in_specs=[
                pl.BlockSpec((_R, 8), lambda b, meta: (b, 0)),
                pl.BlockSpec((8, m), lambda b, meta: (0, 0)),
            ],
            out_specs=[
                pl.BlockSpec((_R, _KPAD), lambda b, meta: (b, 0)),
            ],
            scratch_shapes=[pltpu.VMEM((_R, -(-m // _T) * _T), jnp.float32),
                            pltpu.VMEM((_R, -(-m // _T) * _T), jnp.float32)],
        ),
        out_shape=[
            jax.ShapeDtypeStruct((m, _KPAD), jnp.int32),
        ],
        compiler_params=pltpu.CompilerParams(
            dimension_semantics=("parallel",)),
        interpret=interpret,
    )(meta, rows_arr, colst)


def _window_meta(batch, m):
    bfirst = batch[:: _R]
    blast = batch[_R - 1 :: _R]
    lo = jnp.searchsorted(batch, bfirst, side="left").astype(jnp.int32)
    hi = jnp.searchsorted(batch, blast, side="right").astype(jnp.int32)
    lo_a = (lo // 128) * 128
    nt = (hi - lo_a + _T - 1) // _T
    return jnp.stack([lo_a, nt], axis=1).astype(jnp.int32)


def kernel(x, pos, edge_index, edge_weight, batch, perm, score, i):
    m = perm.shape[0]

    table_pad = jnp.pad(pos, ((0, 0), (0, _DPAD - pos.shape[1])))
    idx_grouped = perm.reshape(32, -1, 128)
    pos_p16 = _sc_gather_rows(table_pad, idx_grouped, m)
    pos_p = pos_p16[:, :3]

    batch_f = batch.astype(jnp.float32)
    sq = jnp.sum(pos_p * pos_p, axis=1)
    rows_arr = jnp.concatenate(
        [pos_p, batch_f[:, None], sq[:, None], jnp.zeros((m, 3), jnp.float32)],
        axis=1,
    )
    colst = rows_arr.T
    meta = _window_meta(batch, m)

    (idx_out,) = _tc_knn_topk(meta, rows_arr, colst, m)
    nbr = idx_out[:, :_K]

    src = nbr.reshape(-1)
    dst = jnp.repeat(jnp.arange(m, dtype=jnp.int32), _K)
    ei_new = jnp.stack([src, dst], axis=0)
    d2e = _sc_edge_d2(pos_p[:, 0], pos_p[:, 1], pos_p[:, 2], src, dst)
    dist = jnp.sqrt(jnp.maximum(d2e, jnp.float32(1e-12)))
    ew_new = dist / jnp.maximum(jnp.max(dist), jnp.float32(1e-12))
    return (x, pos_p, ei_new, ew_new, batch, perm, score)
